# Initial kernel scaffold; baseline (speedup 1.0000x reference)
import jax
import jax.numpy as jnp
from jax.experimental import pallas as pl


def kernel(values, tables):
    raise NotImplementedError("write your pallas kernel here")


# speedup vs baseline: 3.5505x; 3.5505x over previous
import jax
import jax.numpy as jnp
from jax import lax
from jax.experimental import pallas as pl
from jax.experimental.pallas import tpu as pltpu
from jax.experimental.pallas import tpu_sc as plsc

_T = 26
_V = 100000
_D = 32
_B = 81920
_NC = 2
_NS = 16
_NW = _NC * _NS
_CHUNK = 128
_ROWS = _B // _NW // _CHUNK


def _body(values_hbm, tables_hbm, out_hbm, idx_v, rows_v, gsem):
    wid = lax.axis_index("s") * _NC + lax.axis_index("c")

    def table_step(t, carry):
        pltpu.sync_copy(values_hbm.at[t, wid], idx_v)
        offv = jnp.full((16,), t * _V, dtype=jnp.int32)

        def add_off(j, c2):
            for i in range(_CHUNK // 16):
                sl = pl.ds(i * 16, 16)
                idx_v[j, sl] = idx_v[j, sl] + offv
            return c2

        lax.fori_loop(0, _ROWS, add_off, 0)

        copies = [
            pltpu.async_copy(tables_hbm.at[idx_v.at[j]], rows_v.at[j], gsem)
            for j in range(_ROWS)
        ]
        for cp in copies:
            cp.wait()
        pltpu.sync_copy(rows_v, out_hbm.at[t, wid])
        return carry

    lax.fori_loop(0, _T, table_step, 0)


def kernel(values, tables):
    tables_flat = tables.reshape(_T * _V, _D)
    values_r = values.reshape(_T, _NW, _ROWS, _CHUNK)
    mesh = plsc.VectorSubcoreMesh(core_axis_name="c", subcore_axis_name="s")
    out = pl.kernel(
        _body,
        out_type=jax.ShapeDtypeStruct((_T, _NW, _ROWS, _CHUNK, _D), jnp.float32),
        mesh=mesh,
        scratch_types=[
            pltpu.VMEM((_ROWS, _CHUNK), jnp.int32),
            pltpu.VMEM((_ROWS, _CHUNK, _D), jnp.float32),
            pltpu.SemaphoreType.DMA,
        ],
        compiler_params=pltpu.CompilerParams(use_tc_tiling_on_sc=False),
    )(values_r, tables_flat)
    return out.reshape(_T, _B, _D)


# speedup vs baseline: 3.5560x; 1.0015x over previous
import jax
import jax.numpy as jnp
from jax import lax
from jax.experimental import pallas as pl
from jax.experimental.pallas import tpu as pltpu
from jax.experimental.pallas import tpu_sc as plsc

_T = 26
_V = 100000
_D = 32
_B = 81920
_NC = 2
_NS = 16
_NW = _NC * _NS
_CHUNK = 128
_ROWS = _B // _NW // _CHUNK


def _body(values_hbm, tables_hbm, out_hbm, idx_v, rows_v, gsem):
    wid = lax.axis_index("s") * _NC + lax.axis_index("c")

    def table_step(t, carry):
        pltpu.sync_copy(values_hbm.at[t, wid], idx_v)
        copies = [
            pltpu.async_copy(
                tables_hbm.at[t].at[idx_v.at[j]],
                rows_v.at[pl.ds(j * _CHUNK, _CHUNK)],
                gsem,
            )
            for j in range(_ROWS)
        ]
        for cp in copies:
            cp.wait()
        pltpu.sync_copy(rows_v, out_hbm.at[t, pl.ds(wid * _ROWS * _CHUNK, _ROWS * _CHUNK)])
        return carry

    lax.fori_loop(0, _T, table_step, 0)


def kernel(values, tables):
    values_r = values.reshape(_T, _NW, _ROWS, _CHUNK)
    mesh = plsc.VectorSubcoreMesh(core_axis_name="c", subcore_axis_name="s")
    out = pl.kernel(
        _body,
        out_type=jax.ShapeDtypeStruct((_T, _B, _D), jnp.float32),
        mesh=mesh,
        scratch_types=[
            pltpu.VMEM((_ROWS, _CHUNK), jnp.int32),
            pltpu.VMEM((_ROWS * _CHUNK, _D), jnp.float32),
            pltpu.SemaphoreType.DMA,
        ],
        compiler_params=pltpu.CompilerParams(use_tc_tiling_on_sc=False),
    )(values_r, tables)
    return out
